# Initial kernel scaffold; baseline (speedup 1.0000x reference)
import jax
import jax.numpy as jnp
from jax.experimental import pallas as pl


def kernel(x, edge_index, W, b):
    raise NotImplementedError("write your pallas kernel here")


# speedup vs baseline: 14.1637x; 14.1637x over previous
import jax
import jax.numpy as jnp
from jax import lax
from jax.experimental import pallas as pl
from jax.experimental.pallas import tpu as pltpu
from jax.experimental.pallas import tpu_sc as plsc

N = 100000
E = 1600000
DIN = 512
DOUT = 32
HOPS = 3

NBLK = E // 128
SB = 50
NSUPER = NBLK // SB
RB = 2000
GRID = N // RB
RCH = 250
NRCH = N // RCH

_f32 = jnp.float32
_mesh = plsc.VectorSubcoreMesh(
    core_axis_name="c", subcore_axis_name="s", num_cores=2, num_subcores=16)
_params = pltpu.CompilerParams(use_tc_tiling_on_sc=False)


def _fill(ref, n, value):
    @pl.loop(0, n)
    def _(i):
        ref[i] = jnp.full((16,), value, _f32)


def _row_chunks(s, fn):
    @pl.loop(s, NRCH, step=16)
    def _(ck):
        fn(ck * RCH)


def _deg_body(col_ref, deg_ref, acc, colbuf, ones, zbuf):
    c = lax.axis_index("c")
    s = lax.axis_index("s")
    wid = s * 2 + c
    _fill(ones, 128, 1.0)
    _fill(zbuf, RCH, 0.0)
    _row_chunks(s, lambda off: pltpu.sync_copy(zbuf, acc.at[pl.ds(off, RCH)]))
    plsc.subcore_barrier()

    @pl.loop(wid, NSUPER, step=32)
    def _(sidx):
        pltpu.sync_copy(col_ref.at[sidx], colbuf)

        @pl.loop(0, SB)
        def _(j):
            pltpu.sync_copy(ones, acc.at[colbuf.at[j]], add=True)

    plsc.subcore_barrier()
    _row_chunks(s, lambda off: pltpu.sync_copy(
        acc.at[pl.ds(off, RCH)], deg_ref.at[c].at[pl.ds(off, RCH)]))


_deg_call = pl.kernel(
    _deg_body,
    out_type=jax.ShapeDtypeStruct((2, N, 16), _f32),
    mesh=_mesh,
    scratch_types=[
        pltpu.VMEM_SHARED((N, 16), _f32),
        pltpu.VMEM((SB, 128), jnp.int32),
        pltpu.VMEM((128, 16), _f32),
        pltpu.VMEM((RCH, 16), _f32),
    ],
    compiler_params=_params,
)


def _hop_body(g_ref, row_ref, col_ref, s_ref,
              acc, rowbuf, colbuf, zbuf, rb0, rb1, rb2, rb3,
              sem0, sem1, sem2, sem3):
    c = lax.axis_index("c")
    s = lax.axis_index("s")
    rbufs = (rb0, rb1, rb2, rb3)
    sems = (sem0, sem1, sem2, sem3)
    gc = g_ref.at[c]

    _fill(zbuf, RCH, 0.0)
    _row_chunks(s, lambda off: pltpu.sync_copy(zbuf, acc.at[pl.ds(off, RCH)]))
    plsc.subcore_barrier()

    @pl.loop(s, NSUPER, step=16)
    def _(sidx):
        pltpu.sync_copy(row_ref.at[sidx], rowbuf)
        pltpu.sync_copy(col_ref.at[sidx], colbuf)

        @pl.loop(0, SB)
        def _(j):
            pltpu.sync_copy(gc.at[rowbuf.at[j]], rb0)
            pltpu.sync_copy(rb0, acc.at[colbuf.at[j]], add=True)

    plsc.subcore_barrier()
    _row_chunks(s, lambda off: pltpu.sync_copy(
        acc.at[pl.ds(off, RCH)], s_ref.at[c].at[pl.ds(off, RCH)]))


_hop_call = pl.kernel(
    _hop_body,
    out_type=jax.ShapeDtypeStruct((2, N, 16), _f32),
    mesh=_mesh,
    scratch_types=[
        pltpu.VMEM_SHARED((N, 16), _f32),
        pltpu.VMEM((SB, 128), jnp.int32),
        pltpu.VMEM((SB, 128), jnp.int32),
        pltpu.VMEM((RCH, 16), _f32),
        pltpu.VMEM((128, 16), _f32),
        pltpu.VMEM((128, 16), _f32),
        pltpu.VMEM((128, 16), _f32),
        pltpu.VMEM((128, 16), _f32),
        pltpu.SemaphoreType.DMA,
        pltpu.SemaphoreType.DMA,
        pltpu.SemaphoreType.DMA,
        pltpu.SemaphoreType.DMA,
    ],
    compiler_params=_params,
)


def _dis_of(d_ref):
    d = d_ref[0, :, 0:1] + d_ref[1, :, 0:1]
    pos = d > 0
    dsafe = jnp.where(pos, d, 1.0)
    return pos, dsafe


def _proj_body(x_ref, w_ref, b_ref, d_ref, g_ref):
    pos, dsafe = _dis_of(d_ref)
    dis = jnp.where(pos, lax.rsqrt(dsafe), 0.0)
    xb = x_ref[...]
    h0 = lax.dot_general(xb, w_ref[0:16, :], (((1,), (1,)), ((), ())),
                         preferred_element_type=_f32)
    h1 = lax.dot_general(xb, w_ref[16:32, :], (((1,), (1,)), ((), ())),
                         preferred_element_type=_f32)
    g_ref[0, :, :] = (h0 + b_ref[0:1, 0:16]) * dis
    g_ref[1, :, :] = (h1 + b_ref[0:1, 16:32]) * dis


_proj_call = pl.pallas_call(
    _proj_body,
    grid=(GRID,),
    in_specs=[
        pl.BlockSpec((RB, DIN), lambda i: (i, 0)),
        pl.BlockSpec((DOUT, DIN), lambda i: (0, 0)),
        pl.BlockSpec((1, DOUT), lambda i: (0, 0)),
        pl.BlockSpec((2, RB, 16), lambda i: (0, i, 0)),
    ],
    out_specs=pl.BlockSpec((2, RB, 16), lambda i: (0, i, 0)),
    out_shape=jax.ShapeDtypeStruct((2, N, 16), _f32),
)


def _scale_body(s_ref, d_ref, o_ref):
    pos, dsafe = _dis_of(d_ref)
    dis2 = jnp.where(pos, 1.0 / dsafe, 0.0)
    o_ref[0, :, :] = s_ref[0, :, :] * dis2
    o_ref[1, :, :] = s_ref[1, :, :] * dis2


_scale_call = pl.pallas_call(
    _scale_body,
    grid=(GRID,),
    in_specs=[
        pl.BlockSpec((2, RB, 16), lambda i: (0, i, 0)),
        pl.BlockSpec((2, RB, 16), lambda i: (0, i, 0)),
    ],
    out_specs=pl.BlockSpec((2, RB, 16), lambda i: (0, i, 0)),
    out_shape=jax.ShapeDtypeStruct((2, N, 16), _f32),
)


def _final_body(s_ref, d_ref, o_ref):
    pos, dsafe = _dis_of(d_ref)
    dis = jnp.where(pos, lax.rsqrt(dsafe), 0.0)
    o_ref[...] = jnp.concatenate(
        [s_ref[0, :, :] * dis, s_ref[1, :, :] * dis], axis=1)


_final_call = pl.pallas_call(
    _final_body,
    grid=(GRID,),
    in_specs=[
        pl.BlockSpec((2, RB, 16), lambda i: (0, i, 0)),
        pl.BlockSpec((2, RB, 16), lambda i: (0, i, 0)),
    ],
    out_specs=pl.BlockSpec((RB, DOUT), lambda i: (i, 0)),
    out_shape=jax.ShapeDtypeStruct((N, DOUT), _f32),
)


def kernel(x, edge_index, W, b):
    row_r = edge_index[0].reshape(NSUPER, SB, 128)
    col_r = edge_index[1].reshape(NSUPER, SB, 128)
    deg = _deg_call(col_r)
    g = _proj_call(x, W, b.reshape(1, DOUT), deg)
    for hop in range(HOPS):
        s = _hop_call(g, row_r, col_r)
        if hop < HOPS - 1:
            g = _scale_call(s, deg)
    return _final_call(s, deg)


# speedup vs baseline: 20.3854x; 1.4393x over previous
import jax
import jax.numpy as jnp
from jax import lax
from jax.experimental import pallas as pl
from jax.experimental.pallas import tpu as pltpu
from jax.experimental.pallas import tpu_sc as plsc

N = 100000
E = 1600000
DIN = 512
DOUT = 32
HOPS = 3

NBLK = E // 128
SB = 25
NSUPER = NBLK // SB
RB = 2000
GRID = N // RB
RCH = 125
NRCH = N // RCH

_f32 = jnp.float32
_mesh = plsc.VectorSubcoreMesh(
    core_axis_name="c", subcore_axis_name="s", num_cores=2, num_subcores=16)
_params = pltpu.CompilerParams(use_tc_tiling_on_sc=False)


def _fill(ref, n, value):
    @pl.loop(0, n)
    def _(i):
        ref[i] = jnp.full((16,), value, _f32)


def _row_chunks(s, fn):
    @pl.loop(s, NRCH, step=16)
    def _(ck):
        fn(ck * RCH)


def _deg_body(col_ref, deg_ref, acc, colbuf, ones, zbuf):
    c = lax.axis_index("c")
    s = lax.axis_index("s")
    wid = s * 2 + c
    _fill(ones, 128, 1.0)
    _fill(zbuf, RCH, 0.0)
    _row_chunks(s, lambda off: pltpu.sync_copy(zbuf, acc.at[pl.ds(off, RCH)]))
    plsc.subcore_barrier()

    @pl.loop(wid, NSUPER, step=32)
    def _(sidx):
        pltpu.sync_copy(col_ref.at[sidx], colbuf)

        @pl.loop(0, SB)
        def _(j):
            pltpu.sync_copy(ones, acc.at[colbuf.at[j]], add=True)

    plsc.subcore_barrier()
    _row_chunks(s, lambda off: pltpu.sync_copy(
        acc.at[pl.ds(off, RCH)], deg_ref.at[c].at[pl.ds(off, RCH)]))


_deg_call = pl.kernel(
    _deg_body,
    out_type=jax.ShapeDtypeStruct((2, N, 16), _f32),
    mesh=_mesh,
    scratch_types=[
        pltpu.VMEM_SHARED((N, 16), _f32),
        pltpu.VMEM((SB, 128), jnp.int32),
        pltpu.VMEM((128, 16), _f32),
        pltpu.VMEM((RCH, 16), _f32),
    ],
    compiler_params=_params,
)


def _hop_body(g_ref, row_ref, col_ref, s_ref,
              acc, rowbuf, colbuf, zbuf, rb0, rb1, rb2, rb3, rb4,
              sem0):
    c = lax.axis_index("c")
    s = lax.axis_index("s")
    rbufs = (rb0, rb1, rb2, rb3, rb4)
    gc = g_ref.at[c]

    _fill(zbuf, RCH, 0.0)
    _row_chunks(s, lambda off: pltpu.sync_copy(zbuf, acc.at[pl.ds(off, RCH)]))
    plsc.subcore_barrier()

    @pl.loop(s, NSUPER, step=16)
    def _(sidx):
        pltpu.sync_copy(row_ref.at[sidx], rowbuf)
        pltpu.sync_copy(col_ref.at[sidx], colbuf)

        @pl.loop(0, SB, step=5)
        def _(i):
            for bq in range(5):
                pltpu.make_async_copy(
                    gc.at[rowbuf.at[i + bq]], rbufs[bq], sem0).start()
            for bq in range(5):
                pltpu.make_async_copy(
                    gc.at[rowbuf.at[i + bq]], rbufs[bq], sem0).wait()
            for bq in range(5):
                pltpu.sync_copy(rbufs[bq], acc.at[colbuf.at[i + bq]], add=True)

    plsc.subcore_barrier()
    _row_chunks(s, lambda off: pltpu.sync_copy(
        acc.at[pl.ds(off, RCH)], s_ref.at[c].at[pl.ds(off, RCH)]))


_hop_call = pl.kernel(
    _hop_body,
    out_type=jax.ShapeDtypeStruct((2, N, 16), _f32),
    mesh=_mesh,
    scratch_types=[
        pltpu.VMEM_SHARED((N, 16), _f32),
        pltpu.VMEM((SB, 128), jnp.int32),
        pltpu.VMEM((SB, 128), jnp.int32),
        pltpu.VMEM((RCH, 16), _f32),
        pltpu.VMEM((128, 16), _f32),
        pltpu.VMEM((128, 16), _f32),
        pltpu.VMEM((128, 16), _f32),
        pltpu.VMEM((128, 16), _f32),
        pltpu.VMEM((128, 16), _f32),
        pltpu.SemaphoreType.DMA,
    ],
    compiler_params=_params,
)


def _dis_of(d_ref):
    d = d_ref[0, :, 0:1] + d_ref[1, :, 0:1]
    pos = d > 0
    dsafe = jnp.where(pos, d, 1.0)
    return pos, dsafe


def _proj_body(x_ref, w_ref, b_ref, d_ref, g_ref):
    pos, dsafe = _dis_of(d_ref)
    dis = jnp.where(pos, lax.rsqrt(dsafe), 0.0)
    xb = x_ref[...]
    h0 = lax.dot_general(xb, w_ref[0:16, :], (((1,), (1,)), ((), ())),
                         preferred_element_type=_f32)
    h1 = lax.dot_general(xb, w_ref[16:32, :], (((1,), (1,)), ((), ())),
                         preferred_element_type=_f32)
    g_ref[0, :, :] = (h0 + b_ref[0:1, 0:16]) * dis
    g_ref[1, :, :] = (h1 + b_ref[0:1, 16:32]) * dis


_proj_call = pl.pallas_call(
    _proj_body,
    grid=(GRID,),
    in_specs=[
        pl.BlockSpec((RB, DIN), lambda i: (i, 0)),
        pl.BlockSpec((DOUT, DIN), lambda i: (0, 0)),
        pl.BlockSpec((1, DOUT), lambda i: (0, 0)),
        pl.BlockSpec((2, RB, 16), lambda i: (0, i, 0)),
    ],
    out_specs=pl.BlockSpec((2, RB, 16), lambda i: (0, i, 0)),
    out_shape=jax.ShapeDtypeStruct((2, N, 16), _f32),
)


def _scale_body(s_ref, d_ref, o_ref):
    pos, dsafe = _dis_of(d_ref)
    dis2 = jnp.where(pos, 1.0 / dsafe, 0.0)
    o_ref[0, :, :] = s_ref[0, :, :] * dis2
    o_ref[1, :, :] = s_ref[1, :, :] * dis2


_scale_call = pl.pallas_call(
    _scale_body,
    grid=(GRID,),
    in_specs=[
        pl.BlockSpec((2, RB, 16), lambda i: (0, i, 0)),
        pl.BlockSpec((2, RB, 16), lambda i: (0, i, 0)),
    ],
    out_specs=pl.BlockSpec((2, RB, 16), lambda i: (0, i, 0)),
    out_shape=jax.ShapeDtypeStruct((2, N, 16), _f32),
)


def _final_body(s_ref, d_ref, o_ref):
    pos, dsafe = _dis_of(d_ref)
    dis = jnp.where(pos, lax.rsqrt(dsafe), 0.0)
    o_ref[...] = jnp.concatenate(
        [s_ref[0, :, :] * dis, s_ref[1, :, :] * dis], axis=1)


_final_call = pl.pallas_call(
    _final_body,
    grid=(GRID,),
    in_specs=[
        pl.BlockSpec((2, RB, 16), lambda i: (0, i, 0)),
        pl.BlockSpec((2, RB, 16), lambda i: (0, i, 0)),
    ],
    out_specs=pl.BlockSpec((RB, DOUT), lambda i: (i, 0)),
    out_shape=jax.ShapeDtypeStruct((N, DOUT), _f32),
)


def kernel(x, edge_index, W, b):
    row_r = edge_index[0].reshape(NSUPER, SB, 128)
    col_r = edge_index[1].reshape(NSUPER, SB, 128)
    deg = _deg_call(col_r)
    g = _proj_call(x, W, b.reshape(1, DOUT), deg)
    for hop in range(HOPS):
        s = _hop_call(g, row_r, col_r)
        if hop < HOPS - 1:
            g = _scale_call(s, deg)
    return _final_call(s, deg)


# speedup vs baseline: 25.5323x; 1.2525x over previous
import jax
import jax.numpy as jnp
from jax import lax
from jax.experimental import pallas as pl
from jax.experimental.pallas import tpu as pltpu
from jax.experimental.pallas import tpu_sc as plsc

N = 100000
E = 1600000
DIN = 512
DOUT = 32
HOPS = 3

NBLK = E // 128
SB = 25
NSUPER = NBLK // SB
RB = 2000
GRID = N // RB
RCH = 125
NRCH = N // RCH

_f32 = jnp.float32
_mesh = plsc.VectorSubcoreMesh(
    core_axis_name="c", subcore_axis_name="s", num_cores=2, num_subcores=16)
_params = pltpu.CompilerParams(use_tc_tiling_on_sc=False)


def _fill(ref, n, value):
    @pl.loop(0, n)
    def _(i):
        ref[i] = jnp.full((16,), value, _f32)


def _row_chunks(s, fn):
    @pl.loop(s, NRCH, step=16)
    def _(ck):
        fn(ck * RCH)


def _row_chunks_async(s, mk, sem):
    @pl.loop(s, NRCH, step=16)
    def _(ck):
        mk(ck * RCH).start()

    @pl.loop(s, NRCH, step=16)
    def _(ck):
        mk(ck * RCH).wait()


def _deg_body(col_ref, deg_ref, acc, colbuf, ones, zbuf):
    c = lax.axis_index("c")
    s = lax.axis_index("s")
    wid = s * 2 + c
    _fill(ones, 128, 1.0)
    _fill(zbuf, RCH, 0.0)
    _row_chunks(s, lambda off: pltpu.sync_copy(zbuf, acc.at[pl.ds(off, RCH)]))
    plsc.subcore_barrier()

    @pl.loop(wid, NSUPER, step=32)
    def _(sidx):
        pltpu.sync_copy(col_ref.at[sidx], colbuf)

        @pl.loop(0, SB)
        def _(j):
            pltpu.sync_copy(ones, acc.at[colbuf.at[j]], add=True)

    plsc.subcore_barrier()
    _row_chunks(s, lambda off: pltpu.sync_copy(
        acc.at[pl.ds(off, RCH)], deg_ref.at[c].at[pl.ds(off, RCH)]))


_deg_call = pl.kernel(
    _deg_body,
    out_type=jax.ShapeDtypeStruct((2, N, 16), _f32),
    mesh=_mesh,
    scratch_types=[
        pltpu.VMEM_SHARED((N, 16), _f32),
        pltpu.VMEM((SB, 128), jnp.int32),
        pltpu.VMEM((128, 16), _f32),
        pltpu.VMEM((RCH, 16), _f32),
    ],
    compiler_params=_params,
)


def _hop_body(g_ref, ei_ref, s_ref,
              acc, eibuf, zbuf,
              ra0, ra1, ra2, ra3, ra4, rb0, rb1, rb2, rb3, rb4,
              sema, semb, semz):
    c = lax.axis_index("c")
    s = lax.axis_index("s")
    banks = ((ra0, ra1, ra2, ra3, ra4), (rb0, rb1, rb2, rb3, rb4))
    sems = (sema, semb)
    gc = g_ref.at[c]
    rowidx = eibuf.at[0]
    colidx = eibuf.at[1]

    _fill(zbuf, RCH, 0.0)
    _row_chunks_async(
        s, lambda off: pltpu.make_async_copy(
            zbuf, acc.at[pl.ds(off, RCH)], semz), semz)
    plsc.subcore_barrier()

    def _fire(gi, bk):
        for bq in range(5):
            pltpu.make_async_copy(
                gc.at[rowidx.at[gi * 5 + bq]], banks[bk][bq], sems[bk]).start()

    def _drain_scatter(gi, bk):
        for bq in range(5):
            pltpu.make_async_copy(
                gc.at[rowidx.at[gi * 5 + bq]], banks[bk][bq], sems[bk]).wait()
        for bq in range(5):
            pltpu.sync_copy(
                banks[bk][bq], acc.at[colidx.at[gi * 5 + bq]], add=True)

    @pl.loop(s, NSUPER, step=16)
    def _(sidx):
        pltpu.sync_copy(ei_ref.at[sidx], eibuf)
        _fire(0, 0)
        for gi in range(SB // 5):
            if gi + 1 < SB // 5:
                _fire(gi + 1, (gi + 1) % 2)
            _drain_scatter(gi, gi % 2)

    plsc.subcore_barrier()
    _row_chunks_async(
        s, lambda off: pltpu.make_async_copy(
            acc.at[pl.ds(off, RCH)], s_ref.at[c].at[pl.ds(off, RCH)], semz),
        semz)


_hop_call = pl.kernel(
    _hop_body,
    out_type=jax.ShapeDtypeStruct((2, N, 16), _f32),
    mesh=_mesh,
    scratch_types=[
        pltpu.VMEM_SHARED((N, 16), _f32),
        pltpu.VMEM((2, SB, 128), jnp.int32),
        pltpu.VMEM((RCH, 16), _f32),
    ] + [pltpu.VMEM((128, 16), _f32)] * 10 + [
        pltpu.SemaphoreType.DMA,
        pltpu.SemaphoreType.DMA,
        pltpu.SemaphoreType.DMA,
    ],
    compiler_params=_params,
)


def _dis_of(d_ref):
    d = d_ref[0, :, 0:1] + d_ref[1, :, 0:1]
    pos = d > 0
    dsafe = jnp.where(pos, d, 1.0)
    return pos, dsafe


def _proj_body(x_ref, w_ref, b_ref, d_ref, g_ref):
    pos, dsafe = _dis_of(d_ref)
    dis = jnp.where(pos, lax.rsqrt(dsafe), 0.0)
    xb = x_ref[...]
    h0 = lax.dot_general(xb, w_ref[0:16, :], (((1,), (1,)), ((), ())),
                         preferred_element_type=_f32)
    h1 = lax.dot_general(xb, w_ref[16:32, :], (((1,), (1,)), ((), ())),
                         preferred_element_type=_f32)
    g_ref[0, :, :] = (h0 + b_ref[0:1, 0:16]) * dis
    g_ref[1, :, :] = (h1 + b_ref[0:1, 16:32]) * dis


_proj_call = pl.pallas_call(
    _proj_body,
    grid=(GRID,),
    in_specs=[
        pl.BlockSpec((RB, DIN), lambda i: (i, 0)),
        pl.BlockSpec((DOUT, DIN), lambda i: (0, 0)),
        pl.BlockSpec((1, DOUT), lambda i: (0, 0)),
        pl.BlockSpec((2, RB, 16), lambda i: (0, i, 0)),
    ],
    out_specs=pl.BlockSpec((2, RB, 16), lambda i: (0, i, 0)),
    out_shape=jax.ShapeDtypeStruct((2, N, 16), _f32),
)


def _scale_body(s_ref, d_ref, o_ref):
    pos, dsafe = _dis_of(d_ref)
    dis2 = jnp.where(pos, 1.0 / dsafe, 0.0)
    o_ref[0, :, :] = s_ref[0, :, :] * dis2
    o_ref[1, :, :] = s_ref[1, :, :] * dis2


_scale_call = pl.pallas_call(
    _scale_body,
    grid=(GRID,),
    in_specs=[
        pl.BlockSpec((2, RB, 16), lambda i: (0, i, 0)),
        pl.BlockSpec((2, RB, 16), lambda i: (0, i, 0)),
    ],
    out_specs=pl.BlockSpec((2, RB, 16), lambda i: (0, i, 0)),
    out_shape=jax.ShapeDtypeStruct((2, N, 16), _f32),
)


def _final_body(s_ref, d_ref, o_ref):
    pos, dsafe = _dis_of(d_ref)
    dis = jnp.where(pos, lax.rsqrt(dsafe), 0.0)
    o_ref[...] = jnp.concatenate(
        [s_ref[0, :, :] * dis, s_ref[1, :, :] * dis], axis=1)


_final_call = pl.pallas_call(
    _final_body,
    grid=(GRID,),
    in_specs=[
        pl.BlockSpec((2, RB, 16), lambda i: (0, i, 0)),
        pl.BlockSpec((2, RB, 16), lambda i: (0, i, 0)),
    ],
    out_specs=pl.BlockSpec((RB, DOUT), lambda i: (i, 0)),
    out_shape=jax.ShapeDtypeStruct((N, DOUT), _f32),
)


def kernel(x, edge_index, W, b):
    col_r = edge_index[1].reshape(NSUPER, SB, 128)
    ei_r = edge_index.reshape(2, NSUPER, SB, 128).transpose(1, 0, 2, 3)
    deg = _deg_call(col_r)
    g = _proj_call(x, W, b.reshape(1, DOUT), deg)
    for hop in range(HOPS):
        s = _hop_call(g, ei_r)
        if hop < HOPS - 1:
            g = _scale_call(s, deg)
    return _final_call(s, deg)
